# baseline (speedup 1.0000x reference)
import jax
import jax.numpy as jnp
from jax.experimental import pallas as pl

B = 256
E = 62
TOP_K = 10


def _conv(x, w, b):
    y = jax.lax.conv_general_dilated(x, w, window_strides=(1, 1), padding='VALID',
                                     dimension_numbers=('NCHW', 'OIHW', 'NCHW'))
    return y + b[None, :, None, None]


def _maxpool_1x2(x):
    return jax.lax.reduce_window(x, -jnp.inf, jax.lax.max, (1, 1, 1, 2), (1, 1, 1, 2), 'VALID')


def _sogc(H, Wbn, Wg, bg):
    G = jnp.tanh(jnp.einsum('bed,kd->bek', H, Wbn))
    A = jax.nn.softmax(jnp.einsum('bek,bfk->bef', G, G), axis=-1)
    vals, idxs = jax.lax.top_k(A, TOP_K)
    Bb, Ee, _ = A.shape
    bi = jnp.arange(Bb)[:, None, None]
    ei = jnp.arange(Ee)[None, :, None]
    A_sparse = jnp.zeros_like(A).at[bi, ei, idxs].set(vals)
    H_agg = jnp.einsum('bef,bfd->bed', A_sparse, H)
    return jax.nn.relu(jnp.einsum('bed,od->beo', H_agg, Wg) + bg)


def _fc_kernel(h_ref, w_ref, b_ref, o_ref):
    o_ref[...] = jax.lax.dot_general(
        h_ref[...], w_ref[...], (((1,), (1,)), ((), ())),
        preferred_element_type=jnp.float32) + b_ref[...]


def kernel(x, Wc1, bc1, Wc2, bc2, Wc3, bc3, Wbn1, Wg1, bg1, Wbn2, Wg2, bg2, Wbn3, Wg3, bg3, Wfc, bfc):
    h = x.reshape(B * E, 1, x.shape[2], x.shape[3])
    h = jax.nn.relu(_conv(h, Wc1, bc1))
    h = _maxpool_1x2(h)
    h = jax.nn.relu(_conv(h, Wc2, bc2))
    h = _maxpool_1x2(h)
    h = jax.nn.relu(_conv(h, Wc3, bc3))
    h = _maxpool_1x2(h)
    h = h.reshape(B, E, -1)
    h = _sogc(h, Wbn1, Wg1, bg1)
    h = _sogc(h, Wbn2, Wg2, bg2)
    h = _sogc(h, Wbn3, Wg3, bg3)
    h = h.reshape(B, -1)
    logits = pl.pallas_call(
        _fc_kernel,
        out_shape=jax.ShapeDtypeStruct((B, 4), jnp.float32),
    )(h, Wfc, bfc.reshape(1, 4))
    return logits


# speedup vs baseline: 1.2372x; 1.2372x over previous
import jax
import jax.numpy as jnp
from jax.experimental import pallas as pl
from jax.experimental.pallas import tpu as pltpu

B = 256
E = 62
TOP_K = 10

P_BLK = 496
BS = 8


def _bdot(a, b):
    return jax.lax.dot_general(
        a.astype(jnp.bfloat16), b.astype(jnp.bfloat16),
        (((1,), (0,)), ((), ())), preferred_element_type=jnp.float32)


def _bdot_t(a, b):
    return jax.lax.dot_general(
        a.astype(jnp.bfloat16), b.astype(jnp.bfloat16),
        (((1,), (1,)), ((), ())), preferred_element_type=jnp.float32)


def _conv_kernel(x_ref, t1e_ref, t1o_ref, b1_ref, t2e_ref, t2o_ref, b2_ref,
                 t3e_ref, t3o_ref, b3_ref, out_ref):
    x = x_ref[...]
    p1 = jax.nn.relu(jnp.maximum(_bdot(x, t1e_ref[...]), _bdot(x, t1o_ref[...]))
                     + b1_ref[...])
    p2 = jax.nn.relu(jnp.maximum(_bdot(p1, t2e_ref[...]), _bdot(p1, t2o_ref[...]))
                     + b2_ref[...])
    p3 = jax.nn.relu(jnp.maximum(_bdot(p2, t3e_ref[...]), _bdot(p2, t3o_ref[...]))
                     + b3_ref[...])
    out_ref[...] = p3


def _sogc_layer(H, WbnT, WgT, bg):
    G = jnp.tanh(_bdot(H, WbnT))
    S = _bdot_t(G, G)
    m = jnp.max(S, axis=1, keepdims=True)
    ex = jnp.exp(S - m)
    A = ex / jnp.sum(ex, axis=1, keepdims=True)
    ii = jax.lax.broadcasted_iota(jnp.int32, (E, E), 1)
    work = A
    mask = jnp.zeros((E, E), dtype=jnp.float32)
    for _ in range(TOP_K):
        mx = jnp.max(work, axis=1, keepdims=True)
        eq = work == mx
        idx = jnp.min(jnp.where(eq, ii, E + 1), axis=1, keepdims=True)
        onehot = ii == idx
        work = jnp.where(onehot, -1.0, work)
        mask = mask + onehot.astype(jnp.float32)
    As = A * mask
    H_agg = _bdot(As, H)
    return jax.nn.relu(_bdot(H_agg, WgT) + bg)


def _sogc_kernel(h_ref, wbn1_ref, wg1_ref, bg1_ref, wbn2_ref, wg2_ref, bg2_ref,
                 wbn3_ref, wg3_ref, bg3_ref, out_ref):
    for s in range(BS):
        H = h_ref[s]
        H1 = _sogc_layer(H, wbn1_ref[...], wg1_ref[...], bg1_ref[...])
        H2 = _sogc_layer(H1, wbn2_ref[...], wg2_ref[...], bg2_ref[...])
        H3 = _sogc_layer(H2, wbn3_ref[...], wg3_ref[...], bg3_ref[...])
        out_ref[s] = H3


def _fc_kernel(h_ref, w_ref, b_ref, o_ref):
    o_ref[...] = _bdot_t(h_ref[...], w_ref[...]) + b_ref[...]


def _shift_sel(n_in, n_out, w, phase):
    u = jnp.arange(n_in)[:, None, None]
    t = jnp.arange(n_out)[None, :, None]
    ww = jnp.arange(w)[None, None, :]
    return (u == 2 * t + phase + ww).astype(jnp.float32)


def _toeplitz1(W1, phase):
    S = _shift_sel(64, 30, 5, phase)
    T = jnp.einsum('utw,chw->hutc', S, W1[:, 0])
    return T.reshape(320, 960)


def _toeplitz2(W2, phase):
    S = _shift_sel(30, 13, 5, phase)
    T = jnp.einsum('abw,oiw->aibo', S, W2[:, :, 0, :])
    return T.reshape(960, 832)


def _toeplitz3(W3, phase):
    S = _shift_sel(13, 4, 5, phase)
    T = jnp.einsum('abw,oiw->aibo', S, W3[:, :, 0, :])
    return T.reshape(832, 512)


def kernel(x, Wc1, bc1, Wc2, bc2, Wc3, bc3, Wbn1, Wg1, bg1, Wbn2, Wg2, bg2,
           Wbn3, Wg3, bg3, Wfc, bfc):
    bf = jnp.bfloat16
    x_flat = x.reshape(B * E, 320)
    t1e = _toeplitz1(Wc1, 0).astype(bf)
    t1o = _toeplitz1(Wc1, 1).astype(bf)
    t2e = _toeplitz2(Wc2, 0).astype(bf)
    t2o = _toeplitz2(Wc2, 1).astype(bf)
    t3e = _toeplitz3(Wc3, 0).astype(bf)
    t3o = _toeplitz3(Wc3, 1).astype(bf)
    b1 = jnp.tile(bc1, 30).reshape(1, 960)
    b2 = jnp.tile(bc2, 13).reshape(1, 832)
    b3 = jnp.tile(bc3, 4).reshape(1, 512)

    n_blocks = (B * E) // P_BLK
    feats = pl.pallas_call(
        _conv_kernel,
        grid=(n_blocks,),
        in_specs=[
            pl.BlockSpec((P_BLK, 320), lambda i: (i, 0)),
            pl.BlockSpec((320, 960), lambda i: (0, 0)),
            pl.BlockSpec((320, 960), lambda i: (0, 0)),
            pl.BlockSpec((1, 960), lambda i: (0, 0)),
            pl.BlockSpec((960, 832), lambda i: (0, 0)),
            pl.BlockSpec((960, 832), lambda i: (0, 0)),
            pl.BlockSpec((1, 832), lambda i: (0, 0)),
            pl.BlockSpec((832, 512), lambda i: (0, 0)),
            pl.BlockSpec((832, 512), lambda i: (0, 0)),
            pl.BlockSpec((1, 512), lambda i: (0, 0)),
        ],
        out_specs=pl.BlockSpec((P_BLK, 512), lambda i: (i, 0)),
        out_shape=jax.ShapeDtypeStruct((B * E, 512), jnp.float32),
        compiler_params=pltpu.CompilerParams(
            dimension_semantics=("parallel",)),
    )(x_flat, t1e, t1o, b1, t2e, t2o, b2, t3e, t3o, b3)

    j = jnp.arange(512)
    perm = (j % 128) * 4 + j // 128
    H = feats.reshape(B, E, 512)

    wbn1 = Wbn1[:, perm].T.astype(bf)
    wg1 = Wg1[:, perm].T.astype(bf)
    h3 = pl.pallas_call(
        _sogc_kernel,
        grid=(B // BS,),
        in_specs=[
            pl.BlockSpec((BS, E, 512), lambda i: (i, 0, 0)),
            pl.BlockSpec((512, 32), lambda i: (0, 0)),
            pl.BlockSpec((512, 64), lambda i: (0, 0)),
            pl.BlockSpec((1, 64), lambda i: (0, 0)),
            pl.BlockSpec((64, 32), lambda i: (0, 0)),
            pl.BlockSpec((64, 64), lambda i: (0, 0)),
            pl.BlockSpec((1, 64), lambda i: (0, 0)),
            pl.BlockSpec((64, 32), lambda i: (0, 0)),
            pl.BlockSpec((64, 64), lambda i: (0, 0)),
            pl.BlockSpec((1, 64), lambda i: (0, 0)),
        ],
        out_specs=pl.BlockSpec((BS, E, 64), lambda i: (i, 0, 0)),
        out_shape=jax.ShapeDtypeStruct((B, E, 64), jnp.float32),
        compiler_params=pltpu.CompilerParams(
            dimension_semantics=("parallel",)),
    )(H, wbn1, wg1, bg1.reshape(1, 64),
      Wbn2.T.astype(bf), Wg2.T.astype(bf), bg2.reshape(1, 64),
      Wbn3.T.astype(bf), Wg3.T.astype(bf), bg3.reshape(1, 64))

    hf = h3.reshape(B, E * 64)
    logits = pl.pallas_call(
        _fc_kernel,
        out_shape=jax.ShapeDtypeStruct((B, 4), jnp.float32),
    )(hf, Wfc, bfc.reshape(1, 4))
    return logits


# speedup vs baseline: 6.3563x; 5.1377x over previous
import jax
import jax.numpy as jnp
from jax.experimental import pallas as pl
from jax.experimental.pallas import tpu as pltpu

B = 256
E = 62
TOP_K = 10

P_BLK = 496
BS = 8


def _bdot(a, b):
    return jax.lax.dot_general(
        a.astype(jnp.bfloat16), b.astype(jnp.bfloat16),
        (((1,), (0,)), ((), ())), preferred_element_type=jnp.float32)


def _bdot_t(a, b):
    return jax.lax.dot_general(
        a.astype(jnp.bfloat16), b.astype(jnp.bfloat16),
        (((1,), (1,)), ((), ())), preferred_element_type=jnp.float32)


def _bdot3(a, b):
    return jax.lax.dot_general(
        a.astype(jnp.bfloat16), b.astype(jnp.bfloat16),
        (((2,), (0,)), ((), ())), preferred_element_type=jnp.float32)


def _bdot_bt(a, b):
    return jax.lax.dot_general(
        a.astype(jnp.bfloat16), b.astype(jnp.bfloat16),
        (((2,), (2,)), ((0,), (0,))), preferred_element_type=jnp.float32)


def _bdot_b1(a, b):
    return jax.lax.dot_general(
        a.astype(jnp.bfloat16), b.astype(jnp.bfloat16),
        (((1,), (1,)), ((0,), (0,))), preferred_element_type=jnp.float32)


def _conv_kernel(x_ref, t1e_ref, t1o_ref, b1_ref, t2e_ref, t2o_ref, b2_ref,
                 t3e_ref, t3o_ref, b3_ref, out_ref):
    x = x_ref[...]
    p1 = jax.nn.relu(jnp.maximum(_bdot(x, t1e_ref[...]), _bdot(x, t1o_ref[...]))
                     + b1_ref[...])
    p2 = jax.nn.relu(jnp.maximum(_bdot(p1, t2e_ref[...]), _bdot(p1, t2o_ref[...]))
                     + b2_ref[...])
    p3 = jax.nn.relu(jnp.maximum(_bdot(p2, t3e_ref[...]), _bdot(p2, t3o_ref[...]))
                     + b3_ref[...])
    out_ref[...] = p3


def _sogc_layer(H, WbnT, WgT, bg):
    G = jnp.tanh(_bdot3(H, WbnT))
    S = _bdot_bt(G, G)
    m = jnp.max(S, axis=1, keepdims=True)
    ex = jnp.exp(S - m)
    A = ex / jnp.sum(ex, axis=1, keepdims=True)
    bs = A.shape[0]
    fi = jax.lax.broadcasted_iota(jnp.int32, (bs, E, E), 1)
    work = A
    mask = jnp.zeros(A.shape, dtype=jnp.float32)
    for _ in range(TOP_K):
        mx = jnp.max(work, axis=1, keepdims=True)
        eq = work == mx
        idx = jnp.min(jnp.where(eq, fi, E), axis=1, keepdims=True)
        onehot = fi == idx
        work = jnp.where(onehot, -1.0, work)
        mask = mask + onehot.astype(jnp.float32)
    As = A * mask
    H_agg = _bdot_b1(As, H)
    return jax.nn.relu(_bdot3(H_agg, WgT) + bg)


def _sogc_kernel(h_ref, wbn1_ref, wg1_ref, bg1_ref, wbn2_ref, wg2_ref, bg2_ref,
                 wbn3_ref, wg3_ref, bg3_ref, out_ref):
    H = h_ref[...]
    H1 = _sogc_layer(H, wbn1_ref[...], wg1_ref[...], bg1_ref[...])
    H2 = _sogc_layer(H1, wbn2_ref[...], wg2_ref[...], bg2_ref[...])
    H3 = _sogc_layer(H2, wbn3_ref[...], wg3_ref[...], bg3_ref[...])
    out_ref[...] = H3


def _fc_kernel(h_ref, w_ref, b_ref, o_ref):
    o_ref[...] = _bdot_t(h_ref[...], w_ref[...]) + b_ref[...]


def _shift_sel(n_in, n_out, w, phase):
    u = jnp.arange(n_in)[:, None, None]
    t = jnp.arange(n_out)[None, :, None]
    ww = jnp.arange(w)[None, None, :]
    return (u == 2 * t + phase + ww).astype(jnp.float32)


def _toeplitz1(W1, phase):
    S = _shift_sel(64, 30, 5, phase)
    T = jnp.einsum('utw,chw->hutc', S, W1[:, 0])
    return T.reshape(320, 960)


def _toeplitz2(W2, phase):
    S = _shift_sel(30, 13, 5, phase)
    T = jnp.einsum('abw,oiw->aibo', S, W2[:, :, 0, :])
    return T.reshape(960, 832)


def _toeplitz3(W3, phase):
    S = _shift_sel(13, 4, 5, phase)
    T = jnp.einsum('abw,oiw->aibo', S, W3[:, :, 0, :])
    return T.reshape(832, 512)


def kernel(x, Wc1, bc1, Wc2, bc2, Wc3, bc3, Wbn1, Wg1, bg1, Wbn2, Wg2, bg2,
           Wbn3, Wg3, bg3, Wfc, bfc):
    bf = jnp.bfloat16
    x_flat = x.reshape(B * E, 320)
    t1e = _toeplitz1(Wc1, 0).astype(bf)
    t1o = _toeplitz1(Wc1, 1).astype(bf)
    t2e = _toeplitz2(Wc2, 0).astype(bf)
    t2o = _toeplitz2(Wc2, 1).astype(bf)
    t3e = _toeplitz3(Wc3, 0).astype(bf)
    t3o = _toeplitz3(Wc3, 1).astype(bf)
    b1 = jnp.tile(bc1, 30).reshape(1, 960)
    b2 = jnp.tile(bc2, 13).reshape(1, 832)
    b3 = jnp.tile(bc3, 4).reshape(1, 512)

    n_blocks = (B * E) // P_BLK
    feats = pl.pallas_call(
        _conv_kernel,
        grid=(n_blocks,),
        in_specs=[
            pl.BlockSpec((P_BLK, 320), lambda i: (i, 0)),
            pl.BlockSpec((320, 960), lambda i: (0, 0)),
            pl.BlockSpec((320, 960), lambda i: (0, 0)),
            pl.BlockSpec((1, 960), lambda i: (0, 0)),
            pl.BlockSpec((960, 832), lambda i: (0, 0)),
            pl.BlockSpec((960, 832), lambda i: (0, 0)),
            pl.BlockSpec((1, 832), lambda i: (0, 0)),
            pl.BlockSpec((832, 512), lambda i: (0, 0)),
            pl.BlockSpec((832, 512), lambda i: (0, 0)),
            pl.BlockSpec((1, 512), lambda i: (0, 0)),
        ],
        out_specs=pl.BlockSpec((P_BLK, 512), lambda i: (i, 0)),
        out_shape=jax.ShapeDtypeStruct((B * E, 512), jnp.float32),
        compiler_params=pltpu.CompilerParams(
            dimension_semantics=("parallel",)),
    )(x_flat, t1e, t1o, b1, t2e, t2o, b2, t3e, t3o, b3)

    j = jnp.arange(512)
    perm = (j % 128) * 4 + j // 128
    H = feats.reshape(B, E, 512)

    wbn1 = Wbn1[:, perm].T.astype(bf)
    wg1 = Wg1[:, perm].T.astype(bf)
    h3 = pl.pallas_call(
        _sogc_kernel,
        grid=(B // BS,),
        in_specs=[
            pl.BlockSpec((BS, E, 512), lambda i: (i, 0, 0)),
            pl.BlockSpec((512, 32), lambda i: (0, 0)),
            pl.BlockSpec((512, 64), lambda i: (0, 0)),
            pl.BlockSpec((1, 1, 64), lambda i: (0, 0, 0)),
            pl.BlockSpec((64, 32), lambda i: (0, 0)),
            pl.BlockSpec((64, 64), lambda i: (0, 0)),
            pl.BlockSpec((1, 1, 64), lambda i: (0, 0, 0)),
            pl.BlockSpec((64, 32), lambda i: (0, 0)),
            pl.BlockSpec((64, 64), lambda i: (0, 0)),
            pl.BlockSpec((1, 1, 64), lambda i: (0, 0, 0)),
        ],
        out_specs=pl.BlockSpec((BS, E, 64), lambda i: (i, 0, 0)),
        out_shape=jax.ShapeDtypeStruct((B, E, 64), jnp.float32),
        compiler_params=pltpu.CompilerParams(
            dimension_semantics=("parallel",)),
    )(H, wbn1, wg1, bg1.reshape(1, 1, 64),
      Wbn2.T.astype(bf), Wg2.T.astype(bf), bg2.reshape(1, 1, 64),
      Wbn3.T.astype(bf), Wg3.T.astype(bf), bg3.reshape(1, 1, 64))

    hf = h3.reshape(B, E * 64)
    logits = pl.pallas_call(
        _fc_kernel,
        out_shape=jax.ShapeDtypeStruct((B, 4), jnp.float32),
    )(hf, Wfc, bfc.reshape(1, 4))
    return logits


# speedup vs baseline: 6.9766x; 1.0976x over previous
import jax
import jax.numpy as jnp
from jax.experimental import pallas as pl
from jax.experimental.pallas import tpu as pltpu

B = 256
E = 62
TOP_K = 10

P_BLK = 496
BS = 8


def _bdot(a, b):
    return jax.lax.dot_general(
        a.astype(jnp.bfloat16), b.astype(jnp.bfloat16),
        (((1,), (0,)), ((), ())), preferred_element_type=jnp.float32)


def _bdot_t(a, b):
    return jax.lax.dot_general(
        a.astype(jnp.bfloat16), b.astype(jnp.bfloat16),
        (((1,), (1,)), ((), ())), preferred_element_type=jnp.float32)


def _bdot3(a, b):
    return jax.lax.dot_general(
        a, b, (((2,), (0,)), ((), ())), preferred_element_type=jnp.float32)


def _bdot_bt(a, b):
    return jax.lax.dot_general(
        a, b, (((2,), (2,)), ((0,), (0,))), preferred_element_type=jnp.float32)


def _bdot_b1(a, b):
    return jax.lax.dot_general(
        a, b, (((1,), (1,)), ((0,), (0,))), preferred_element_type=jnp.float32)


def _conv_kernel(x_ref, t1e_ref, t1o_ref, b1_ref, t2e_ref, t2o_ref, b2_ref,
                 t3e_ref, t3o_ref, b3_ref, out_ref):
    x = x_ref[...]
    p1 = jax.nn.relu(jnp.maximum(_bdot(x, t1e_ref[...]), _bdot(x, t1o_ref[...]))
                     + b1_ref[...])
    p2 = jax.nn.relu(jnp.maximum(_bdot(p1, t2e_ref[...]), _bdot(p1, t2o_ref[...]))
                     + b2_ref[...])
    p3 = jax.nn.relu(jnp.maximum(_bdot(p2, t3e_ref[...]), _bdot(p2, t3o_ref[...]))
                     + b3_ref[...])
    out_ref[...] = p3


def _sogc_layer(Hb, WbnT, WgT, bg):
    G = jnp.tanh(_bdot3(Hb, WbnT))
    Gb = G.astype(jnp.bfloat16)
    S = _bdot_bt(Gb, Gb)
    hb = S.shape[0] // 2
    Sp = jnp.concatenate([S[:hb], S[hb:]], axis=2)
    m = jnp.max(Sp, axis=1, keepdims=True)
    ex = jnp.exp(Sp - m)
    A = ex * (1.0 / jnp.sum(ex, axis=1, keepdims=True))
    fdesc = jax.lax.broadcasted_iota(
        jnp.int32, (hb, E, 2 * E), 1).astype(jnp.float32) * -1.0
    work = A
    for _ in range(TOP_K):
        mx = jnp.max(work, axis=1, keepdims=True)
        key = jnp.where(work == mx, fdesc, -jnp.inf)
        sel = jnp.max(key, axis=1, keepdims=True)
        work = jnp.where(key == sel, -1.0, work)
    Asp = jnp.where(work < 0.0, A, 0.0)
    As = jnp.concatenate([Asp[:, :, :E], Asp[:, :, E:]], axis=0)
    H_agg = _bdot_b1(As.astype(jnp.bfloat16), Hb)
    return jax.nn.relu(_bdot3(H_agg.astype(jnp.bfloat16), WgT) + bg)


def _sogc_block(H, wbn1, wg1, bg1, wbn2, wg2, bg2, wbn3, wg3, bg3):
    H1 = _sogc_layer(H.astype(jnp.bfloat16), wbn1, wg1, bg1)
    H2 = _sogc_layer(H1.astype(jnp.bfloat16), wbn2, wg2, bg2)
    return _sogc_layer(H2.astype(jnp.bfloat16), wbn3, wg3, bg3)


def _sogc_kernel(h_ref, wbn1_ref, wg1_ref, bg1_ref, wbn2_ref, wg2_ref, bg2_ref,
                 wbn3_ref, wg3_ref, bg3_ref, out_ref):
    out_ref[...] = _sogc_block(
        h_ref[...], wbn1_ref[...], wg1_ref[...], bg1_ref[...],
        wbn2_ref[...], wg2_ref[...], bg2_ref[...],
        wbn3_ref[...], wg3_ref[...], bg3_ref[...])


def _fc_kernel(h_ref, w_ref, b_ref, o_ref):
    o_ref[...] = _bdot_t(h_ref[...], w_ref[...]) + b_ref[...]


def _shift_sel(n_in, n_out, w, phase):
    u = jnp.arange(n_in)[:, None, None]
    t = jnp.arange(n_out)[None, :, None]
    ww = jnp.arange(w)[None, None, :]
    return (u == 2 * t + phase + ww).astype(jnp.float32)


def _toeplitz1(W1, phase):
    S = _shift_sel(64, 30, 5, phase)
    T = jnp.einsum('utw,chw->hutc', S, W1[:, 0])
    return T.reshape(320, 960)


def _toeplitz2(W2, phase):
    S = _shift_sel(30, 13, 5, phase)
    T = jnp.einsum('abw,oiw->aibo', S, W2[:, :, 0, :])
    return T.reshape(960, 832)


def _toeplitz3(W3, phase):
    S = _shift_sel(13, 4, 5, phase)
    T = jnp.einsum('abw,oiw->aibo', S, W3[:, :, 0, :])
    return T.reshape(832, 512)


def kernel(x, Wc1, bc1, Wc2, bc2, Wc3, bc3, Wbn1, Wg1, bg1, Wbn2, Wg2, bg2,
           Wbn3, Wg3, bg3, Wfc, bfc):
    bf = jnp.bfloat16
    x_flat = x.reshape(B * E, 320)
    t1e = _toeplitz1(Wc1, 0).astype(bf)
    t1o = _toeplitz1(Wc1, 1).astype(bf)
    t2e = _toeplitz2(Wc2, 0).astype(bf)
    t2o = _toeplitz2(Wc2, 1).astype(bf)
    t3e = _toeplitz3(Wc3, 0).astype(bf)
    t3o = _toeplitz3(Wc3, 1).astype(bf)
    b1 = jnp.tile(bc1, 30).reshape(1, 960)
    b2 = jnp.tile(bc2, 13).reshape(1, 832)
    b3 = jnp.tile(bc3, 4).reshape(1, 512)

    n_blocks = (B * E) // P_BLK
    feats = pl.pallas_call(
        _conv_kernel,
        grid=(n_blocks,),
        in_specs=[
            pl.BlockSpec((P_BLK, 320), lambda i: (i, 0)),
            pl.BlockSpec((320, 960), lambda i: (0, 0)),
            pl.BlockSpec((320, 960), lambda i: (0, 0)),
            pl.BlockSpec((1, 960), lambda i: (0, 0)),
            pl.BlockSpec((960, 832), lambda i: (0, 0)),
            pl.BlockSpec((960, 832), lambda i: (0, 0)),
            pl.BlockSpec((1, 832), lambda i: (0, 0)),
            pl.BlockSpec((832, 512), lambda i: (0, 0)),
            pl.BlockSpec((832, 512), lambda i: (0, 0)),
            pl.BlockSpec((1, 512), lambda i: (0, 0)),
        ],
        out_specs=pl.BlockSpec((P_BLK, 512), lambda i: (i, 0)),
        out_shape=jax.ShapeDtypeStruct((B * E, 512), jnp.float32),
        compiler_params=pltpu.CompilerParams(
            dimension_semantics=("parallel",)),
    )(x_flat, t1e, t1o, b1, t2e, t2o, b2, t3e, t3o, b3)

    j = jnp.arange(512)
    perm = (j % 128) * 4 + j // 128
    H = feats.reshape(B, E, 512)

    wbn1 = Wbn1[:, perm].T.astype(bf)
    wg1 = Wg1[:, perm].T.astype(bf)
    h3 = pl.pallas_call(
        _sogc_kernel,
        grid=(B // BS,),
        in_specs=[
            pl.BlockSpec((BS, E, 512), lambda i: (i, 0, 0)),
            pl.BlockSpec((512, 32), lambda i: (0, 0)),
            pl.BlockSpec((512, 64), lambda i: (0, 0)),
            pl.BlockSpec((1, 1, 64), lambda i: (0, 0, 0)),
            pl.BlockSpec((64, 32), lambda i: (0, 0)),
            pl.BlockSpec((64, 64), lambda i: (0, 0)),
            pl.BlockSpec((1, 1, 64), lambda i: (0, 0, 0)),
            pl.BlockSpec((64, 32), lambda i: (0, 0)),
            pl.BlockSpec((64, 64), lambda i: (0, 0)),
            pl.BlockSpec((1, 1, 64), lambda i: (0, 0, 0)),
        ],
        out_specs=pl.BlockSpec((BS, E, 64), lambda i: (i, 0, 0)),
        out_shape=jax.ShapeDtypeStruct((B, E, 64), jnp.float32),
        compiler_params=pltpu.CompilerParams(
            dimension_semantics=("parallel",)),
    )(H, wbn1, wg1, bg1.reshape(1, 1, 64),
      Wbn2.T.astype(bf), Wg2.T.astype(bf), bg2.reshape(1, 1, 64),
      Wbn3.T.astype(bf), Wg3.T.astype(bf), bg3.reshape(1, 1, 64))

    hf = h3.reshape(B, E * 64)
    logits = pl.pallas_call(
        _fc_kernel,
        out_shape=jax.ShapeDtypeStruct((B, 4), jnp.float32),
    )(hf, Wfc, bfc.reshape(1, 4))
    return logits


# speedup vs baseline: 7.3727x; 1.0568x over previous
import jax
import jax.numpy as jnp
from jax.experimental import pallas as pl
from jax.experimental.pallas import tpu as pltpu

B = 256
E = 62
TOP_K = 10

P_BLK = 496
BS = 8


def _bdot(a, b):
    return jax.lax.dot_general(
        a.astype(jnp.bfloat16), b.astype(jnp.bfloat16),
        (((1,), (0,)), ((), ())), preferred_element_type=jnp.float32)


def _bdot_t(a, b):
    return jax.lax.dot_general(
        a.astype(jnp.bfloat16), b.astype(jnp.bfloat16),
        (((1,), (1,)), ((), ())), preferred_element_type=jnp.float32)


def _bdot3(a, b):
    return jax.lax.dot_general(
        a, b, (((2,), (0,)), ((), ())), preferred_element_type=jnp.float32)


def _bdot_bt(a, b):
    return jax.lax.dot_general(
        a, b, (((2,), (2,)), ((0,), (0,))), preferred_element_type=jnp.float32)


def _bdot_b1(a, b):
    return jax.lax.dot_general(
        a, b, (((1,), (1,)), ((0,), (0,))), preferred_element_type=jnp.float32)


def _conv_kernel(x_ref, t1e_ref, t1o_ref, b1_ref, t2e_ref, t2o_ref, b2_ref,
                 t3e_ref, t3o_ref, b3_ref, out_ref):
    x = x_ref[...]
    p1 = jax.nn.relu(jnp.maximum(_bdot(x, t1e_ref[...]), _bdot(x, t1o_ref[...]))
                     + b1_ref[...])
    p2 = jax.nn.relu(jnp.maximum(_bdot(p1, t2e_ref[...]), _bdot(p1, t2o_ref[...]))
                     + b2_ref[...])
    p3 = jax.nn.relu(jnp.maximum(_bdot(p2, t3e_ref[...]), _bdot(p2, t3o_ref[...]))
                     + b3_ref[...])
    out_ref[...] = p3


def _sogc_layer(Hb, WbnT, WgT, bg):
    G = jnp.tanh(_bdot3(Hb, WbnT))
    Gb = G.astype(jnp.bfloat16)
    S = _bdot_bt(Gb, Gb)
    hb = S.shape[0] // 2
    Sp = jnp.concatenate([S[:hb], S[hb:]], axis=2)
    m = jnp.max(Sp, axis=1, keepdims=True)
    ex = jnp.exp(Sp - m)
    A = ex * (1.0 / jnp.sum(ex, axis=1, keepdims=True))
    fdesc = jax.lax.broadcasted_iota(
        jnp.int32, (hb, E, 2 * E), 1).astype(jnp.float32) * -1.0
    work = A
    for _ in range(TOP_K):
        mx = jnp.max(work, axis=1, keepdims=True)
        key = jnp.where(work == mx, fdesc, -jnp.inf)
        sel = jnp.max(key, axis=1, keepdims=True)
        work = jnp.where(key == sel, -1.0, work)
    Asp = jnp.where(work < 0.0, A, 0.0)
    As = jnp.concatenate([Asp[:, :, :E], Asp[:, :, E:]], axis=0)
    H_agg = _bdot_b1(As.astype(jnp.bfloat16), Hb)
    return jax.nn.relu(_bdot3(H_agg.astype(jnp.bfloat16), WgT) + bg)


def _sogc_block(H, wbn1, wg1, bg1, wbn2, wg2, bg2, wbn3, wg3, bg3):
    H1 = _sogc_layer(H.astype(jnp.bfloat16), wbn1, wg1, bg1)
    H2 = _sogc_layer(H1.astype(jnp.bfloat16), wbn2, wg2, bg2)
    return _sogc_layer(H2.astype(jnp.bfloat16), wbn3, wg3, bg3)


def _fused_kernel(x_ref, t1e_ref, t1o_ref, b1_ref, t2e_ref, t2o_ref, b2_ref,
                  t3e_ref, t3o_ref, b3_ref,
                  wbn1_ref, wg1_ref, bg1_ref, wbn2_ref, wg2_ref, bg2_ref,
                  wbn3_ref, wg3_ref, bg3_ref, out_ref):
    x = x_ref[...]
    p1 = jax.nn.relu(jnp.maximum(_bdot(x, t1e_ref[...]), _bdot(x, t1o_ref[...]))
                     + b1_ref[...])
    p2 = jax.nn.relu(jnp.maximum(_bdot(p1, t2e_ref[...]), _bdot(p1, t2o_ref[...]))
                     + b2_ref[...])
    p3 = jax.nn.relu(jnp.maximum(_bdot(p2, t3e_ref[...]), _bdot(p2, t3o_ref[...]))
                     + b3_ref[...])
    H = p3.reshape(BS, E, 512)
    out_ref[...] = _sogc_block(
        H, wbn1_ref[...], wg1_ref[...], bg1_ref[...],
        wbn2_ref[...], wg2_ref[...], bg2_ref[...],
        wbn3_ref[...], wg3_ref[...], bg3_ref[...])


def _fc_kernel(h_ref, w_ref, b_ref, o_ref):
    o_ref[...] = _bdot_t(h_ref[...], w_ref[...]) + b_ref[...]


def _shift_sel(n_in, n_out, w, phase):
    u = jnp.arange(n_in)[:, None, None]
    t = jnp.arange(n_out)[None, :, None]
    ww = jnp.arange(w)[None, None, :]
    return (u == 2 * t + phase + ww).astype(jnp.float32)


def _toeplitz1(W1, phase):
    S = _shift_sel(64, 30, 5, phase)
    T = jnp.einsum('utw,chw->hutc', S, W1[:, 0])
    return T.reshape(320, 960)


def _toeplitz2(W2, phase):
    S = _shift_sel(30, 13, 5, phase)
    T = jnp.einsum('abw,oiw->aibo', S, W2[:, :, 0, :])
    return T.reshape(960, 832)


def _toeplitz3(W3, phase):
    S = _shift_sel(13, 4, 5, phase)
    T = jnp.einsum('abw,oiw->aibo', S, W3[:, :, 0, :])
    return T.reshape(832, 512)


def kernel(x, Wc1, bc1, Wc2, bc2, Wc3, bc3, Wbn1, Wg1, bg1, Wbn2, Wg2, bg2,
           Wbn3, Wg3, bg3, Wfc, bfc):
    bf = jnp.bfloat16
    x_flat = x.reshape(B * E, 320)
    t1e = _toeplitz1(Wc1, 0).astype(bf)
    t1o = _toeplitz1(Wc1, 1).astype(bf)
    t2e = _toeplitz2(Wc2, 0).astype(bf)
    t2o = _toeplitz2(Wc2, 1).astype(bf)
    t3e = _toeplitz3(Wc3, 0).astype(bf)
    t3o = _toeplitz3(Wc3, 1).astype(bf)
    b1 = jnp.tile(bc1, 30).reshape(1, 960)
    b2 = jnp.tile(bc2, 13).reshape(1, 832)
    b3 = jnp.tile(bc3, 4).reshape(1, 512)

    j = jnp.arange(512)
    perm = (j % 128) * 4 + j // 128
    wbn1 = Wbn1[:, perm].T.astype(bf)
    wg1 = Wg1[:, perm].T.astype(bf)

    n_blocks = (B * E) // P_BLK
    h3 = pl.pallas_call(
        _fused_kernel,
        grid=(n_blocks,),
        in_specs=[
            pl.BlockSpec((P_BLK, 320), lambda i: (i, 0)),
            pl.BlockSpec((320, 960), lambda i: (0, 0)),
            pl.BlockSpec((320, 960), lambda i: (0, 0)),
            pl.BlockSpec((1, 960), lambda i: (0, 0)),
            pl.BlockSpec((960, 832), lambda i: (0, 0)),
            pl.BlockSpec((960, 832), lambda i: (0, 0)),
            pl.BlockSpec((1, 832), lambda i: (0, 0)),
            pl.BlockSpec((832, 512), lambda i: (0, 0)),
            pl.BlockSpec((832, 512), lambda i: (0, 0)),
            pl.BlockSpec((1, 512), lambda i: (0, 0)),
            pl.BlockSpec((512, 32), lambda i: (0, 0)),
            pl.BlockSpec((512, 64), lambda i: (0, 0)),
            pl.BlockSpec((1, 1, 64), lambda i: (0, 0, 0)),
            pl.BlockSpec((64, 32), lambda i: (0, 0)),
            pl.BlockSpec((64, 64), lambda i: (0, 0)),
            pl.BlockSpec((1, 1, 64), lambda i: (0, 0, 0)),
            pl.BlockSpec((64, 32), lambda i: (0, 0)),
            pl.BlockSpec((64, 64), lambda i: (0, 0)),
            pl.BlockSpec((1, 1, 64), lambda i: (0, 0, 0)),
        ],
        out_specs=pl.BlockSpec((BS, E, 64), lambda i: (i, 0, 0)),
        out_shape=jax.ShapeDtypeStruct((B, E, 64), jnp.float32),
        compiler_params=pltpu.CompilerParams(
            dimension_semantics=("parallel",)),
    )(x_flat, t1e, t1o, b1, t2e, t2o, b2, t3e, t3o, b3,
      wbn1, wg1, bg1.reshape(1, 1, 64),
      Wbn2.T.astype(bf), Wg2.T.astype(bf), bg2.reshape(1, 1, 64),
      Wbn3.T.astype(bf), Wg3.T.astype(bf), bg3.reshape(1, 1, 64))

    hf = h3.reshape(B, E * 64)
    logits = pl.pallas_call(
        _fc_kernel,
        out_shape=jax.ShapeDtypeStruct((B, 4), jnp.float32),
    )(hf, Wfc, bfc.reshape(1, 4))
    return logits


# speedup vs baseline: 7.9953x; 1.0844x over previous
import jax
import jax.numpy as jnp
from jax.experimental import pallas as pl
from jax.experimental.pallas import tpu as pltpu

B = 256
E = 62
TOP_K = 10

P_BLK = 992
BS = 16


def _bdot(a, b):
    return jax.lax.dot_general(
        a.astype(jnp.bfloat16), b.astype(jnp.bfloat16),
        (((1,), (0,)), ((), ())), preferred_element_type=jnp.float32)


def _bdot_t(a, b):
    return jax.lax.dot_general(
        a.astype(jnp.bfloat16), b.astype(jnp.bfloat16),
        (((1,), (1,)), ((), ())), preferred_element_type=jnp.float32)


def _bdot3(a, b):
    return jax.lax.dot_general(
        a, b, (((2,), (0,)), ((), ())), preferred_element_type=jnp.float32)


def _bdot_bt(a, b):
    return jax.lax.dot_general(
        a, b, (((2,), (2,)), ((0,), (0,))), preferred_element_type=jnp.float32)


def _bdot_b1(a, b):
    return jax.lax.dot_general(
        a, b, (((1,), (1,)), ((0,), (0,))), preferred_element_type=jnp.float32)


def _conv_kernel(x_ref, t1e_ref, t1o_ref, b1_ref, t2e_ref, t2o_ref, b2_ref,
                 t3e_ref, t3o_ref, b3_ref, out_ref):
    x = x_ref[...]
    p1 = jax.nn.relu(jnp.maximum(_bdot(x, t1e_ref[...]), _bdot(x, t1o_ref[...]))
                     + b1_ref[...])
    p2 = jax.nn.relu(jnp.maximum(_bdot(p1, t2e_ref[...]), _bdot(p1, t2o_ref[...]))
                     + b2_ref[...])
    p3 = jax.nn.relu(jnp.maximum(_bdot(p2, t3e_ref[...]), _bdot(p2, t3o_ref[...]))
                     + b3_ref[...])
    out_ref[...] = p3


def _sogc_layer(Hb, WbnT, WgT, bg):
    G = jnp.tanh(_bdot3(Hb, WbnT))
    Gb = G.astype(jnp.bfloat16)
    S = _bdot_bt(Gb, Gb)
    hb = S.shape[0] // 2
    Sp = jnp.concatenate([S[:hb], S[hb:]], axis=2)
    m = jnp.max(Sp, axis=1, keepdims=True)
    ex = jnp.exp(Sp - m)
    A = ex * (1.0 / jnp.sum(ex, axis=1, keepdims=True))
    fdesc = jax.lax.broadcasted_iota(
        jnp.int32, (hb, E, 2 * E), 1).astype(jnp.float32) * -1.0
    work = A
    for _ in range(TOP_K):
        mx = jnp.max(work, axis=1, keepdims=True)
        key = jnp.where(work == mx, fdesc, -jnp.inf)
        sel = jnp.max(key, axis=1, keepdims=True)
        work = jnp.where(key == sel, -1.0, work)
    Asp = jnp.where(work < 0.0, A, 0.0)
    As = jnp.concatenate([Asp[:, :, :E], Asp[:, :, E:]], axis=0)
    H_agg = _bdot_b1(As.astype(jnp.bfloat16), Hb)
    return jax.nn.relu(_bdot3(H_agg.astype(jnp.bfloat16), WgT) + bg)


def _sogc_block(H, wbn1, wg1, bg1, wbn2, wg2, bg2, wbn3, wg3, bg3):
    H1 = _sogc_layer(H.astype(jnp.bfloat16), wbn1, wg1, bg1)
    H2 = _sogc_layer(H1.astype(jnp.bfloat16), wbn2, wg2, bg2)
    return _sogc_layer(H2.astype(jnp.bfloat16), wbn3, wg3, bg3)


def _fused_kernel(x_ref, t1e_ref, t1o_ref, b1_ref, t2e_ref, t2o_ref, b2_ref,
                  t3e_ref, t3o_ref, b3_ref,
                  wbn1_ref, wg1_ref, bg1_ref, wbn2_ref, wg2_ref, bg2_ref,
                  wbn3_ref, wg3_ref, bg3_ref, out_ref):
    x = x_ref[...]
    p1 = jax.nn.relu(jnp.maximum(_bdot(x, t1e_ref[...]), _bdot(x, t1o_ref[...]))
                     + b1_ref[...])
    p2 = jax.nn.relu(jnp.maximum(_bdot(p1, t2e_ref[...]), _bdot(p1, t2o_ref[...]))
                     + b2_ref[...])
    p3 = jax.nn.relu(jnp.maximum(_bdot(p2, t3e_ref[...]), _bdot(p2, t3o_ref[...]))
                     + b3_ref[...])
    H = p3.reshape(BS, E, 512)
    out_ref[...] = _sogc_block(
        H, wbn1_ref[...], wg1_ref[...], bg1_ref[...],
        wbn2_ref[...], wg2_ref[...], bg2_ref[...],
        wbn3_ref[...], wg3_ref[...], bg3_ref[...])


def _fc_kernel(h_ref, w_ref, b_ref, o_ref):
    o_ref[...] = _bdot_t(h_ref[...], w_ref[...]) + b_ref[...]


def _shift_sel(n_in, n_out, w, phase):
    u = jnp.arange(n_in)[:, None, None]
    t = jnp.arange(n_out)[None, :, None]
    ww = jnp.arange(w)[None, None, :]
    return (u == 2 * t + phase + ww).astype(jnp.float32)


def _toeplitz1(W1, phase):
    S = _shift_sel(64, 30, 5, phase)
    T = jnp.einsum('utw,chw->hutc', S, W1[:, 0])
    return T.reshape(320, 960)


def _toeplitz2(W2, phase):
    S = _shift_sel(30, 13, 5, phase)
    T = jnp.einsum('abw,oiw->aibo', S, W2[:, :, 0, :])
    return T.reshape(960, 832)


def _toeplitz3(W3, phase):
    S = _shift_sel(13, 4, 5, phase)
    T = jnp.einsum('abw,oiw->aibo', S, W3[:, :, 0, :])
    return T.reshape(832, 512)


def kernel(x, Wc1, bc1, Wc2, bc2, Wc3, bc3, Wbn1, Wg1, bg1, Wbn2, Wg2, bg2,
           Wbn3, Wg3, bg3, Wfc, bfc):
    bf = jnp.bfloat16
    x_flat = x.reshape(B * E, 320)
    t1e = _toeplitz1(Wc1, 0).astype(bf)
    t1o = _toeplitz1(Wc1, 1).astype(bf)
    t2e = _toeplitz2(Wc2, 0).astype(bf)
    t2o = _toeplitz2(Wc2, 1).astype(bf)
    t3e = _toeplitz3(Wc3, 0).astype(bf)
    t3o = _toeplitz3(Wc3, 1).astype(bf)
    b1 = jnp.tile(bc1, 30).reshape(1, 960)
    b2 = jnp.tile(bc2, 13).reshape(1, 832)
    b3 = jnp.tile(bc3, 4).reshape(1, 512)

    j = jnp.arange(512)
    perm = (j % 128) * 4 + j // 128
    wbn1 = Wbn1[:, perm].T.astype(bf)
    wg1 = Wg1[:, perm].T.astype(bf)

    n_blocks = (B * E) // P_BLK
    h3 = pl.pallas_call(
        _fused_kernel,
        grid=(n_blocks,),
        in_specs=[
            pl.BlockSpec((P_BLK, 320), lambda i: (i, 0)),
            pl.BlockSpec((320, 960), lambda i: (0, 0)),
            pl.BlockSpec((320, 960), lambda i: (0, 0)),
            pl.BlockSpec((1, 960), lambda i: (0, 0)),
            pl.BlockSpec((960, 832), lambda i: (0, 0)),
            pl.BlockSpec((960, 832), lambda i: (0, 0)),
            pl.BlockSpec((1, 832), lambda i: (0, 0)),
            pl.BlockSpec((832, 512), lambda i: (0, 0)),
            pl.BlockSpec((832, 512), lambda i: (0, 0)),
            pl.BlockSpec((1, 512), lambda i: (0, 0)),
            pl.BlockSpec((512, 32), lambda i: (0, 0)),
            pl.BlockSpec((512, 64), lambda i: (0, 0)),
            pl.BlockSpec((1, 1, 64), lambda i: (0, 0, 0)),
            pl.BlockSpec((64, 32), lambda i: (0, 0)),
            pl.BlockSpec((64, 64), lambda i: (0, 0)),
            pl.BlockSpec((1, 1, 64), lambda i: (0, 0, 0)),
            pl.BlockSpec((64, 32), lambda i: (0, 0)),
            pl.BlockSpec((64, 64), lambda i: (0, 0)),
            pl.BlockSpec((1, 1, 64), lambda i: (0, 0, 0)),
        ],
        out_specs=pl.BlockSpec((BS, E, 64), lambda i: (i, 0, 0)),
        out_shape=jax.ShapeDtypeStruct((B, E, 64), jnp.float32),
        compiler_params=pltpu.CompilerParams(
            dimension_semantics=("parallel",)),
    )(x_flat, t1e, t1o, b1, t2e, t2o, b2, t3e, t3o, b3,
      wbn1, wg1, bg1.reshape(1, 1, 64),
      Wbn2.T.astype(bf), Wg2.T.astype(bf), bg2.reshape(1, 1, 64),
      Wbn3.T.astype(bf), Wg3.T.astype(bf), bg3.reshape(1, 1, 64))

    hf = h3.reshape(B, E * 64)
    logits = pl.pallas_call(
        _fc_kernel,
        out_shape=jax.ShapeDtypeStruct((B, 4), jnp.float32),
    )(hf, Wfc, bfc.reshape(1, 4))
    return logits
